# initial kernel scaffold (unmeasured)
import jax
import jax.numpy as jnp
from jax import lax
from jax.experimental import pallas as pl
from jax.experimental.pallas import tpu as pltpu


def kernel(
    x,
):
    def body(*refs):
        pass

    out_shape = jax.ShapeDtypeStruct(..., jnp.float32)
    return pl.pallas_call(body, out_shape=out_shape)(...)


# baseline (device time: 44564 ns/iter reference)
import jax
import jax.numpy as jnp
from jax import lax
from jax.experimental import pallas as pl
from jax.experimental.pallas import tpu as pltpu

N_DEV = 4


def kernel(x):
    x2 = x.reshape(x.shape[1], x.shape[2])
    m, n = x2.shape

    def body(x_ref, out_ref, comm_ref, send_sems, recv_sems):
        my_pos = lax.axis_index("i")
        left = lax.rem(my_pos - 1 + N_DEV, N_DEV)
        right = lax.rem(my_pos + 1, N_DEV)

        barrier_sem = pltpu.get_barrier_semaphore()
        for nbr in [left, right]:
            pl.semaphore_signal(
                barrier_sem, inc=1,
                device_id=(nbr,), device_id_type=pl.DeviceIdType.MESH,
            )
        pl.semaphore_wait(barrier_sem, 2)

        out_ref[:, :] = x_ref[:, :]
        comm_ref[0, :, :] = x_ref[:, :]

        for h in range(N_DEV - 1):
            send_slot = h % 2
            recv_slot = (h + 1) % 2
            rdma = pltpu.make_async_remote_copy(
                src_ref=comm_ref.at[send_slot],
                dst_ref=comm_ref.at[recv_slot],
                send_sem=send_sems.at[send_slot],
                recv_sem=recv_sems.at[recv_slot],
                device_id=(right,),
                device_id_type=pl.DeviceIdType.MESH,
            )
            rdma.start()
            rdma.wait()
            out_ref[:, :] += comm_ref[recv_slot, :, :]

    return pl.pallas_call(
        body,
        out_shape=jax.ShapeDtypeStruct((m, n), x2.dtype),
        in_specs=[pl.BlockSpec(memory_space=pltpu.VMEM)],
        out_specs=pl.BlockSpec(memory_space=pltpu.VMEM),
        scratch_shapes=[
            pltpu.VMEM((2, m, n), x2.dtype),
            pltpu.SemaphoreType.DMA((2,)),
            pltpu.SemaphoreType.DMA((2,)),
        ],
        compiler_params=pltpu.CompilerParams(collective_id=0),
    )(x2)


# device time: 19041 ns/iter; 2.3404x vs baseline; 2.3404x over previous
import jax
import jax.numpy as jnp
from jax import lax
from jax.experimental import pallas as pl
from jax.experimental.pallas import tpu as pltpu

N_DEV = 4
HALF = 128


def kernel(x):
    x2 = x.reshape(x.shape[1], x.shape[2])
    m, n = x2.shape

    def body(x_ref, out_ref, recv_ref, send_sems, recv_sems):
        me = lax.axis_index("i")
        p1 = me ^ 1
        p2 = 3 - me

        h_a = (me ^ (me // 2)) % 2
        h_b = me // 2

        keep_a = h_a * HALF
        send_a = (1 - h_a) * HALF
        keep_b = 2 * HALF + h_b * HALF
        send_b = 2 * HALF + (1 - h_b) * HALF

        barrier_sem = pltpu.get_barrier_semaphore()
        for nbr in [p1, p2]:
            pl.semaphore_signal(
                barrier_sem, inc=1,
                device_id=(nbr,), device_id_type=pl.DeviceIdType.MESH,
            )
        pl.semaphore_wait(barrier_sem, 2)

        def exchange(slot, src, dst_slot, partner):
            rdma = pltpu.make_async_remote_copy(
                src_ref=src,
                dst_ref=recv_ref.at[dst_slot],
                send_sem=send_sems.at[slot],
                recv_sem=recv_sems.at[slot],
                device_id=(partner,),
                device_id_type=pl.DeviceIdType.MESH,
            )
            rdma.start()
            return rdma

        ra = exchange(0, x_ref.at[pl.ds(send_a, HALF), :], 0, p1)
        rb = exchange(1, x_ref.at[pl.ds(send_b, HALF), :], 1, p2)
        ra.wait()
        out_ref[pl.ds(keep_a, HALF), :] = (
            x_ref[pl.ds(keep_a, HALF), :] + recv_ref[0, :, :]
        )
        rb.wait()
        out_ref[pl.ds(keep_b, HALF), :] = (
            x_ref[pl.ds(keep_b, HALF), :] + recv_ref[1, :, :]
        )

        ra = exchange(2, out_ref.at[pl.ds(keep_a, HALF), :], 2, p2)
        rb = exchange(3, out_ref.at[pl.ds(keep_b, HALF), :], 3, p1)
        ra.wait()
        out_ref[pl.ds(keep_a, HALF), :] += recv_ref[2, :, :]
        rb.wait()
        out_ref[pl.ds(keep_b, HALF), :] += recv_ref[3, :, :]

        ra = exchange(4, out_ref.at[pl.ds(keep_a, HALF), :], 4, p1)
        rb = exchange(5, out_ref.at[pl.ds(keep_b, HALF), :], 5, p2)
        ra.wait()
        out_ref[pl.ds(send_a, HALF), :] = recv_ref[4, :, :]
        rb.wait()
        out_ref[pl.ds(send_b, HALF), :] = recv_ref[5, :, :]

    return pl.pallas_call(
        body,
        out_shape=jax.ShapeDtypeStruct((m, n), x2.dtype),
        in_specs=[pl.BlockSpec(memory_space=pltpu.VMEM)],
        out_specs=pl.BlockSpec(memory_space=pltpu.VMEM),
        scratch_shapes=[
            pltpu.VMEM((6, HALF, n), x2.dtype),
            pltpu.SemaphoreType.DMA((6,)),
            pltpu.SemaphoreType.DMA((6,)),
        ],
        compiler_params=pltpu.CompilerParams(collective_id=0),
    )(x2)


# device time: 17651 ns/iter; 2.5247x vs baseline; 1.0787x over previous
import jax
import jax.numpy as jnp
from jax import lax
from jax.experimental import pallas as pl
from jax.experimental.pallas import tpu as pltpu

N_DEV = 4
HALF = 128
CHUNKS = 2
CH = HALF // CHUNKS
N_EX = 3 * 2 * CHUNKS


def kernel(x):
    x2 = x.reshape(x.shape[1], x.shape[2])
    m, n = x2.shape

    def body(x_ref, out_ref, recv_ref, send_sems, recv_sems):
        me = lax.axis_index("i")
        p1 = me ^ 1
        p2 = 3 - me

        h_a = (me ^ (me // 2)) % 2
        h_b = me // 2

        keep_off = [h_a * HALF, 2 * HALF + h_b * HALF]
        send_off = [(1 - h_a) * HALF, 2 * HALF + (1 - h_b) * HALF]
        partner = [[p1, p2, p1], [p2, p1, p2]]

        barrier_sem = pltpu.get_barrier_semaphore()
        for nbr in [p1, p2]:
            pl.semaphore_signal(
                barrier_sem, inc=1,
                device_id=(nbr,), device_id_type=pl.DeviceIdType.MESH,
            )
        pl.semaphore_wait(barrier_sem, 2)

        def slot(rnd, blk, c):
            return (rnd * 2 + blk) * CHUNKS + c

        def exchange(sl, src, part):
            rdma = pltpu.make_async_remote_copy(
                src_ref=src,
                dst_ref=recv_ref.at[sl],
                send_sem=send_sems.at[sl],
                recv_sem=recv_sems.at[sl],
                device_id=(part,),
                device_id_type=pl.DeviceIdType.MESH,
            )
            rdma.start()
            return rdma

        inflight = {}
        for blk in (0, 1):
            for c in range(CHUNKS):
                src = x_ref.at[pl.ds(send_off[blk] + c * CH, CH), :]
                inflight[(0, blk, c)] = exchange(
                    slot(0, blk, c), src, partner[blk][0]
                )

        for rnd in range(3):
            for blk in (0, 1):
                for c in range(CHUNKS):
                    inflight[(rnd, blk, c)].wait()
                    sl = slot(rnd, blk, c)
                    ko = keep_off[blk] + c * CH
                    if rnd == 0:
                        out_ref[pl.ds(ko, CH), :] = (
                            x_ref[pl.ds(ko, CH), :] + recv_ref[sl, :, :]
                        )
                    elif rnd == 1:
                        out_ref[pl.ds(ko, CH), :] += recv_ref[sl, :, :]
                    else:
                        so = send_off[blk] + c * CH
                        out_ref[pl.ds(so, CH), :] = recv_ref[sl, :, :]
                    if rnd < 2:
                        src = out_ref.at[pl.ds(ko, CH), :]
                        inflight[(rnd + 1, blk, c)] = exchange(
                            slot(rnd + 1, blk, c), src, partner[blk][rnd + 1]
                        )

    return pl.pallas_call(
        body,
        out_shape=jax.ShapeDtypeStruct((m, n), x2.dtype),
        in_specs=[pl.BlockSpec(memory_space=pltpu.VMEM)],
        out_specs=pl.BlockSpec(memory_space=pltpu.VMEM),
        scratch_shapes=[
            pltpu.VMEM((N_EX, CH, n), x2.dtype),
            pltpu.SemaphoreType.DMA((N_EX,)),
            pltpu.SemaphoreType.DMA((N_EX,)),
        ],
        compiler_params=pltpu.CompilerParams(collective_id=0),
    )(x2)
